# baseline (speedup 1.0000x reference)
import functools
import math

import jax
import jax.numpy as jnp
from jax import lax
from jax.experimental import pallas as pl
from jax.experimental.pallas import tpu as pltpu
from jax.experimental.pallas import tpu_sc as plsc

N = 50000
E = 800000
D = 64
L = 3

NW = 32
EW_REAL = 25000
EW = 25600
E_PAD = NW * EW
CH = 800
NCH = EW // CH
RING = 2
NP = 50176
TRASH = NP - 1
RPT = NP // 16
PB = 2000
EB = 2048
PR = CH // 256

_mesh = plsc.VectorSubcoreMesh(core_axis_name="c", subcore_axis_name="s")
_sc_params = pltpu.CompilerParams(use_tc_tiling_on_sc=False)


def _prep_body(x_ref, w_ref, e3_ref, q3r, kr, vr, q3c, kc, vc):
    x = x_ref[...]
    mm = lambda w: lax.dot_general(x, w, (((1,), (0,)), ((), ())),
                                   preferred_element_type=jnp.float32)
    q3r[...] = mm(w_ref[0])[None, :, :] * e3_ref[0][:, None, :]
    kr[...] = mm(w_ref[1])
    vr[...] = mm(w_ref[2])
    q3c[...] = mm(w_ref[3])[None, :, :] * e3_ref[1][:, None, :]
    kc[...] = mm(w_ref[4])
    vc[...] = mm(w_ref[5])


def _prep(x, ws, e3s):
    return pl.pallas_call(
        _prep_body,
        grid=(N // PB,),
        in_specs=[
            pl.BlockSpec((PB, D), lambda i: (i, 0)),
            pl.BlockSpec((6, D, D), lambda i: (0, 0, 0)),
            pl.BlockSpec((2, 3, D), lambda i: (0, 0, 0)),
        ],
        out_specs=[
            pl.BlockSpec((3, PB, D), lambda i: (0, i, 0)),
            pl.BlockSpec((PB, D), lambda i: (i, 0)),
            pl.BlockSpec((PB, D), lambda i: (i, 0)),
            pl.BlockSpec((3, PB, D), lambda i: (0, i, 0)),
            pl.BlockSpec((PB, D), lambda i: (i, 0)),
            pl.BlockSpec((PB, D), lambda i: (i, 0)),
        ],
        out_shape=[
            jax.ShapeDtypeStruct((3, N, D), jnp.float32),
            jax.ShapeDtypeStruct((N, D), jnp.float32),
            jax.ShapeDtypeStruct((N, D), jnp.float32),
            jax.ShapeDtypeStruct((3, N, D), jnp.float32),
            jax.ShapeDtypeStruct((N, D), jnp.float32),
            jax.ShapeDtypeStruct((N, D), jnp.float32),
        ],
    )(x, ws, e3s)


@functools.partial(
    pl.kernel,
    out_type=[jax.ShapeDtypeStruct((E_PAD, D), jnp.float32)] * 6,
    mesh=_mesh,
    compiler_params=_sc_params,
    scratch_types=[
        pltpu.VMEM((EW,), jnp.int32),
        pltpu.VMEM((CH, D), jnp.float32), pltpu.VMEM((CH, D), jnp.float32),
        pltpu.SemaphoreType.DMA, pltpu.SemaphoreType.DMA,
        pltpu.SemaphoreType.DMA, pltpu.SemaphoreType.DMA,
    ],
)
def _gather6(q3r, kr, vr, q3c, kc, vc, idxq_r, idxkv_r, idxq_c, idxkv_c,
             qgr, kgr, vgr, qgc, kgc, vgc,
             idx_res, s0, s1, g0, g1, w0, w1):
    wid = lax.axis_index("s") * 2 + lax.axis_index("c")
    base = wid * EW
    sb = (s0, s1)
    gs = (g0, g1)
    wsm = (w0, w1)

    def phase(table, idx_row, out):
        pltpu.sync_copy(idx_row, idx_res)
        for jj in range(RING - 1):
            pltpu.async_copy(table.at[idx_res.at[pl.ds(jj * CH, CH)]],
                             sb[jj], gs[jj])

        def step(j, first):
            def slot(b_):
                pltpu.make_async_copy(
                    table.at[idx_res.at[pl.ds(0, CH)]], sb[b_], gs[b_]).wait()
                pltpu.async_copy(sb[b_], out.at[pl.ds(base + j * CH, CH)],
                                 wsm[b_])

            def refill(b_):
                if not first:
                    pltpu.make_async_copy(
                        sb[b_], out.at[pl.ds(base, CH)], wsm[b_]).wait()
                pltpu.async_copy(
                    table.at[idx_res.at[pl.ds((j + RING - 1) * CH, CH)]],
                    sb[b_], gs[b_])

            bp = lax.rem(j + RING - 1, RING)

            @pl.when(j + RING - 1 < NCH)
            def _():
                for b_ in range(RING):
                    pl.when(bp == b_)(functools.partial(refill, b_))

            b = lax.rem(j, RING)
            for b_ in range(RING):
                pl.when(b == b_)(functools.partial(slot, b_))

        step(0, True)
        lax.fori_loop(1, NCH, lambda j, c: (step(j, False), c)[1], 0,
                      unroll=False)
        for b_ in range(RING):
            pltpu.make_async_copy(sb[b_], out.at[pl.ds(base, CH)],
                                  wsm[b_]).wait()

    phase(q3r, idxq_r.at[wid], qgr)
    phase(kr, idxkv_r.at[wid], kgr)
    phase(vr, idxkv_r.at[wid], vgr)
    phase(q3c, idxq_c.at[wid], qgc)
    phase(kc, idxkv_c.at[wid], kgc)
    phase(vc, idxkv_c.at[wid], vgc)


def _edge_body(qg_ref, kg_ref, vg_ref, q0_ref, q1_ref, q2_ref, q3_ref,
               p_ref):
    s = jnp.sum(qg_ref[...] * kg_ref[...], axis=1)
    p = jnp.exp(s)
    pv = vg_ref[...] * p[:, None]
    QW = D // 4
    q0_ref[...] = pv[:, 0 * QW:1 * QW]
    q1_ref[...] = pv[:, 1 * QW:2 * QW]
    q2_ref[...] = pv[:, 2 * QW:3 * QW]
    q3_ref[...] = pv[:, 3 * QW:4 * QW]
    p_ref[...] = p.reshape(EB // 256, 256)


def _edge_math(qg, kg, vg):
    return pl.pallas_call(
        _edge_body,
        grid=(E_PAD // EB,),
        in_specs=[pl.BlockSpec((EB, D), lambda i: (i, 0))] * 3,
        out_specs=[pl.BlockSpec((EB, D // 4), lambda i: (i, 0))] * 4 + [
            pl.BlockSpec((EB // 256, 256), lambda i: (i, 0)),
        ],
        out_shape=[jax.ShapeDtypeStruct((E_PAD, D // 4), jnp.float32)] * 4 + [
            jax.ShapeDtypeStruct((E_PAD // 256, 256), jnp.float32),
        ],
    )(qg, kg, vg)


@functools.partial(
    pl.kernel,
    out_type=[jax.ShapeDtypeStruct((2, 2, 4, NP, D // 4), jnp.float32),
              jax.ShapeDtypeStruct((2, 2, NP), jnp.float32)],
    mesh=_mesh,
    compiler_params=_sc_params,
    scratch_types=[
        pltpu.VMEM((EW,), jnp.int32),
        pltpu.VMEM((EW,), jnp.float32),
        pltpu.VMEM((CH, D // 4), jnp.float32),
        pltpu.VMEM((CH, D // 4), jnp.float32),
        pltpu.VMEM_SHARED((NP, D // 4), jnp.float32),
        pltpu.VMEM_SHARED((NP,), jnp.float32),
        pltpu.SemaphoreType.DMA, pltpu.SemaphoreType.DMA,
        pltpu.SemaphoreType.DMA, pltpu.SemaphoreType.DMA,
        pltpu.SemaphoreType.DMA,
    ],
)
def _scatter2(q0_r, q1_r, q2_r, q3_r, p_r, sidx_r,
              q0_c, q1_c, q2_c, q3_c, p_c, sidx_c, z2, z1,
              accs, dens,
              sidx_res, p_res, v0, v1, acc, den,
              l0, l1, a0, a1, dsem):
    cc = lax.axis_index("c")
    t = lax.axis_index("s")
    wid = t * 2 + cc
    vb = (v0, v1)
    lsm = (l0, l1)
    asx = (a0, a1)
    rows = pl.ds(t * RPT, RPT)

    def agg(srcarr, with_den):
        for jj in range(RING - 1):
            pltpu.async_copy(
                srcarr.at[pl.ds(wid * EW + jj * CH, CH)], vb[jj], lsm[jj])

        def step(j, first):
            sl = sidx_res.at[pl.ds(j * CH, CH)]

            def slot(b_):
                pltpu.make_async_copy(
                    srcarr.at[pl.ds(wid * EW, CH)], vb[b_], lsm[b_]).wait()
                pltpu.async_copy(vb[b_], acc.at[sl], asx[b_], add=True)
                if with_den:
                    pltpu.async_copy(p_res.at[pl.ds(j * CH, CH)], den.at[sl],
                                     dsem, add=True)

            def refill(b_):
                if not first:
                    pltpu.make_async_copy(
                        vb[b_], acc.at[sidx_res.at[pl.ds(0, CH)]],
                        asx[b_]).wait()
                pltpu.async_copy(
                    srcarr.at[pl.ds(wid * EW + (j + RING - 1) * CH, CH)],
                    vb[b_], lsm[b_])

            bp = lax.rem(j + RING - 1, RING)

            @pl.when(j + RING - 1 < NCH)
            def _():
                for b_ in range(RING):
                    pl.when(bp == b_)(functools.partial(refill, b_))

            b = lax.rem(j, RING)
            for b_ in range(RING):
                pl.when(b == b_)(functools.partial(slot, b_))

        step(0, True)
        lax.fori_loop(1, NCH, lambda j, c: (step(j, False), c)[1], 0,
                      unroll=False)
        for b_ in range(RING):
            pltpu.make_async_copy(vb[b_], acc.at[sidx_res.at[pl.ds(0, CH)]],
                                  asx[b_]).wait()
        if with_den:
            for _ in range(NCH):
                pltpu.make_async_copy(p_res.at[pl.ds(0, CH)],
                                      den.at[sidx_res.at[pl.ds(0, CH)]],
                                      dsem).wait()

    pltpu.sync_copy(z2, acc.at[rows])
    pltpu.sync_copy(z1, den.at[rows])
    plsc.subcore_barrier()

    for d, (qs, p2, sidx) in enumerate(
            (((q0_r, q1_r, q2_r, q3_r), p_r, sidx_r),
             ((q0_c, q1_c, q2_c, q3_c), p_c, sidx_c))):
        pltpu.sync_copy(sidx.at[wid], sidx_res)
        pltpu.sync_copy(p2.at[wid], p_res)
        for qi in range(4):
            first = qi == 0
            agg(qs[qi], first)
            plsc.subcore_barrier()
            pltpu.sync_copy(acc.at[rows], accs.at[d, cc, qi, rows])
            if first:
                pltpu.sync_copy(den.at[rows], dens.at[d, cc, rows])
                pltpu.sync_copy(z1, den.at[rows])
            pltpu.sync_copy(z2, acc.at[rows])
            plsc.subcore_barrier()


def _pad_idx(a, fill):
    ap = jnp.full((NW, EW), fill, jnp.int32)
    return ap.at[:, :EW_REAL].set(a.reshape(NW, EW_REAL).astype(jnp.int32))


def kernel(params, atoms, edge_index, edge_ids):
    x = params["atom_table"][atoms]
    src, dst = edge_index[0], edge_index[1]
    eid = edge_ids
    inv = 1.0 / math.sqrt(D)

    idxq_r = _pad_idx(eid * N + dst, 0)
    idxq_c = _pad_idx(eid * N + src, 0)
    idxkv_r = _pad_idx(src, 0)
    idxkv_c = _pad_idx(dst, 0)
    sidx_r = _pad_idx(dst, TRASH)
    sidx_c = _pad_idx(src, TRASH)
    z2 = jnp.zeros((RPT, D // 4), jnp.float32)
    z1 = jnp.zeros((RPT,), jnp.float32)

    for l in range(L):
        ws = jnp.stack([
            params["r2c_Wq"][l], params["r2c_Wk"][l], params["r2c_Wv"][l],
            params["c2r_Wq"][l], params["c2r_Wk"][l], params["c2r_Wv"][l],
        ])
        e3s = jnp.stack([
            jnp.tanh(params["edge_table"] @ params["r2c_We"][l]
                     + params["r2c_be"][l]) * inv,
            jnp.tanh(params["edge_table"] @ params["c2r_We"][l]
                     + params["c2r_be"][l]) * inv,
        ])
        q3r, kr, vr, q3c, kc, vc = _prep(x, ws, e3s)
        qgr, kgr, vgr, qgc, kgc, vgc = _gather6(
            q3r.reshape(3 * N, D), kr, vr, q3c.reshape(3 * N, D), kc, vc,
            idxq_r, idxkv_r, idxq_c, idxkv_c)
        pv_r = _edge_math(qgr, kgr, vgr)
        pv_c = _edge_math(qgc, kgc, vgc)
        accs, dens = _scatter2(
            pv_r[0], pv_r[1], pv_r[2], pv_r[3], pv_r[4].reshape(NW, EW),
            sidx_r,
            pv_c[0], pv_c[1], pv_c[2], pv_c[3], pv_c[4].reshape(NW, EW),
            sidx_c, z2, z1)
        outs = []
        for d in range(2):
            num = jnp.concatenate(
                [accs[d, 0, q] + accs[d, 1, q] for q in range(4)],
                axis=1)[:N]
            den = (dens[d, 0] + dens[d, 1])[:N]
            outs.append(num / (den[:, None] + 1e-16))
        h = jnp.concatenate(outs, axis=-1) @ params["ffn_W"][l]
        y = h + x
        mu = jnp.mean(y, axis=-1, keepdims=True)
        var = jnp.var(y, axis=-1, keepdims=True)
        x = (y - mu) / jnp.sqrt(var + 1e-5) * params["ln_g"][l] + params["ln_b"][l]
    return x


# speedup vs baseline: 1.5583x; 1.5583x over previous
import functools
import math

import jax
import jax.numpy as jnp
from jax import lax
from jax.experimental import pallas as pl
from jax.experimental.pallas import tpu as pltpu
from jax.experimental.pallas import tpu_sc as plsc

N = 50000
E = 800000
D = 64
L = 3

NW = 32
EW_REAL = 25000
EW = 25600
E_PAD = NW * EW
CH = 256
NCH = EW // CH
NBLK = NCH // 2
NP = 51200
TRASH = NP - 1
RPT = NP // 16
PB = 2000
EB = 8192

_mesh = plsc.VectorSubcoreMesh(core_axis_name="c", subcore_axis_name="s")
_sc_params = pltpu.CompilerParams(use_tc_tiling_on_sc=False)


def _prep_body(x_ref, w_ref, e3_ref, q3r, kr, vr, q3c, kc, vc):
    x = x_ref[...]
    mm = lambda w: lax.dot_general(x, w, (((1,), (0,)), ((), ())),
                                   preferred_element_type=jnp.float32)
    xq_r = mm(w_ref[0])
    q3r[...] = xq_r[None, :, :] * e3_ref[0][:, None, :]
    kr[...] = mm(w_ref[1])
    vr[...] = mm(w_ref[2])
    xq_c = mm(w_ref[3])
    q3c[...] = xq_c[None, :, :] * e3_ref[1][:, None, :]
    kc[...] = mm(w_ref[4])
    vc[...] = mm(w_ref[5])


def _prep(x, ws, e3s):
    outs = pl.pallas_call(
        _prep_body,
        grid=(N // PB,),
        in_specs=[
            pl.BlockSpec((PB, D), lambda i: (i, 0)),
            pl.BlockSpec((6, D, D), lambda i: (0, 0, 0)),
            pl.BlockSpec((2, 3, D), lambda i: (0, 0, 0)),
        ],
        out_specs=[
            pl.BlockSpec((3, PB, D), lambda i: (0, i, 0)),
            pl.BlockSpec((PB, D), lambda i: (i, 0)),
            pl.BlockSpec((PB, D), lambda i: (i, 0)),
            pl.BlockSpec((3, PB, D), lambda i: (0, i, 0)),
            pl.BlockSpec((PB, D), lambda i: (i, 0)),
            pl.BlockSpec((PB, D), lambda i: (i, 0)),
        ],
        out_shape=[
            jax.ShapeDtypeStruct((3, N, D), jnp.float32),
            jax.ShapeDtypeStruct((N, D), jnp.float32),
            jax.ShapeDtypeStruct((N, D), jnp.float32),
            jax.ShapeDtypeStruct((3, N, D), jnp.float32),
            jax.ShapeDtypeStruct((N, D), jnp.float32),
            jax.ShapeDtypeStruct((N, D), jnp.float32),
        ],
    )(x, ws, e3s)
    return outs


@functools.partial(
    pl.kernel,
    out_type=[jax.ShapeDtypeStruct((E_PAD, D), jnp.float32)] * 3,
    mesh=_mesh,
    compiler_params=_sc_params,
    scratch_types=[
        pltpu.VMEM((CH,), jnp.int32), pltpu.VMEM((CH,), jnp.int32),
        pltpu.VMEM((CH,), jnp.int32), pltpu.VMEM((CH,), jnp.int32),
        pltpu.VMEM((CH, D), jnp.float32), pltpu.VMEM((CH, D), jnp.float32),
        pltpu.VMEM((CH, D), jnp.float32), pltpu.VMEM((CH, D), jnp.float32),
        pltpu.VMEM((CH, D), jnp.float32), pltpu.VMEM((CH, D), jnp.float32),
        pltpu.SemaphoreType.DMA, pltpu.SemaphoreType.DMA,
        pltpu.SemaphoreType.DMA, pltpu.SemaphoreType.DMA,
        pltpu.SemaphoreType.DMA, pltpu.SemaphoreType.DMA,
        pltpu.SemaphoreType.DMA, pltpu.SemaphoreType.DMA,
        pltpu.SemaphoreType.DMA, pltpu.SemaphoreType.DMA,
        pltpu.SemaphoreType.DMA,
    ],
)
def _gather3(q3t, kt, vt, idxq_h, idxkv_h, qg, kg, vg,
             iq0, iq1, ik0, ik1, qb0, qb1, kb0, kb1, vb0, vb1,
             iqs0, iqs1, iks0, iks1, gq0, gq1, gk0, gk1, gv0, gv1, wsem):
    wid = lax.axis_index("s") * 2 + lax.axis_index("c")
    base = wid * EW
    iqb = (iq0, iq1)
    ikb = (ik0, ik1)
    qbb = (qb0, qb1)
    kbb = (kb0, kb1)
    vbb = (vb0, vb1)
    iqs = (iqs0, iqs1)
    iks = (iks0, iks1)
    gqs = (gq0, gq1)
    gks = (gk0, gk1)
    gvs = (gv0, gv1)

    def run_block(blk, drain):
        if drain:
            for b in (0, 1):
                pltpu.make_async_copy(qbb[b], qg.at[pl.ds(base, CH)], wsem).wait()
                pltpu.make_async_copy(kbb[b], kg.at[pl.ds(base, CH)], wsem).wait()
                pltpu.make_async_copy(vbb[b], vg.at[pl.ds(base, CH)], wsem).wait()
        idescs = []
        for b in (0, 1):
            j = blk * 2 + b
            d1 = pltpu.async_copy(idxq_h.at[wid, j], iqb[b], iqs[b])
            d2 = pltpu.async_copy(idxkv_h.at[wid, j], ikb[b], iks[b])
            idescs.append((d1, d2))
        gdescs = []
        for b in (0, 1):
            idescs[b][0].wait()
            idescs[b][1].wait()
            g1 = pltpu.async_copy(q3t.at[iqb[b]], qbb[b], gqs[b])
            g2 = pltpu.async_copy(kt.at[ikb[b]], kbb[b], gks[b])
            g3 = pltpu.async_copy(vt.at[ikb[b]], vbb[b], gvs[b])
            gdescs.append((g1, g2, g3))
        for b in (0, 1):
            j = blk * 2 + b
            for g, buf, out in zip(gdescs[b], (qbb[b], kbb[b], vbb[b]),
                                   (qg, kg, vg)):
                g.wait()
                pltpu.async_copy(buf, out.at[pl.ds(base + j * CH, CH)], wsem)

    run_block(0, False)
    lax.fori_loop(1, NBLK, lambda blk, c: (run_block(blk, True), c)[1], 0,
                  unroll=False)
    for b in (0, 1):
        pltpu.make_async_copy(qbb[b], qg.at[pl.ds(base, CH)], wsem).wait()
        pltpu.make_async_copy(kbb[b], kg.at[pl.ds(base, CH)], wsem).wait()
        pltpu.make_async_copy(vbb[b], vg.at[pl.ds(base, CH)], wsem).wait()


def _edge_body(qg_ref, kg_ref, vg_ref, lo_ref, hi_ref, p_ref):
    s = jnp.sum(qg_ref[...] * kg_ref[...], axis=1)
    p = jnp.exp(s)
    pv = vg_ref[...] * p[:, None]
    lo_ref[...] = pv[:, : D // 2]
    hi_ref[...] = pv[:, D // 2:]
    p_ref[...] = p.reshape(EB // CH, CH)


def _edge_math(qg, kg, vg):
    return pl.pallas_call(
        _edge_body,
        grid=(E_PAD // EB,),
        in_specs=[pl.BlockSpec((EB, D), lambda i: (i, 0))] * 3,
        out_specs=[
            pl.BlockSpec((EB, D // 2), lambda i: (i, 0)),
            pl.BlockSpec((EB, D // 2), lambda i: (i, 0)),
            pl.BlockSpec((EB // CH, CH), lambda i: (i, 0)),
        ],
        out_shape=[
            jax.ShapeDtypeStruct((E_PAD, D // 2), jnp.float32),
            jax.ShapeDtypeStruct((E_PAD, D // 2), jnp.float32),
            jax.ShapeDtypeStruct((E_PAD // CH, CH), jnp.float32),
        ],
    )(qg, kg, vg)


@functools.partial(
    pl.kernel,
    out_type=[jax.ShapeDtypeStruct((2, 2, NP, D // 2), jnp.float32),
              jax.ShapeDtypeStruct((2, NP), jnp.float32)],
    mesh=_mesh,
    compiler_params=_sc_params,
    scratch_types=[
        pltpu.VMEM((CH,), jnp.int32), pltpu.VMEM((CH,), jnp.int32),
        pltpu.VMEM((CH, D // 2), jnp.float32),
        pltpu.VMEM((CH, D // 2), jnp.float32),
        pltpu.VMEM((CH,), jnp.float32), pltpu.VMEM((CH,), jnp.float32),
        pltpu.VMEM_SHARED((NP, D // 2), jnp.float32),
        pltpu.VMEM_SHARED((NP,), jnp.float32),
        pltpu.SemaphoreType.DMA, pltpu.SemaphoreType.DMA,
        pltpu.SemaphoreType.DMA, pltpu.SemaphoreType.DMA,
        pltpu.SemaphoreType.DMA, pltpu.SemaphoreType.DMA,
        pltpu.SemaphoreType.DMA, pltpu.SemaphoreType.DMA,
    ],
)
def _scatter(pv_lo, pv_hi, p2d, sidx_h, z2_h, z1_h, accs, dens,
             sb0, sb1, pvb0, pvb1, pb0, pb1, acc, den,
             ss0, ss1, ls0, ls1, ps0, ps1, ssem, dsem):
    cc = lax.axis_index("c")
    t = lax.axis_index("s")
    wid = t * 2 + cc
    sbb = (sb0, sb1)
    pvb = (pvb0, pvb1)
    pbb = (pb0, pb1)
    sls = (ss0, ss1)
    lls = (ls0, ls1)
    pls = (ps0, ps1)

    pltpu.sync_copy(z2_h, acc.at[pl.ds(t * RPT, RPT)])
    pltpu.sync_copy(z1_h, den.at[pl.ds(t * RPT, RPT)])
    plsc.subcore_barrier()

    def run_block(src, blk, drain, with_den):
        if drain:
            for b in (0, 1):
                pltpu.make_async_copy(pvb[b], acc.at[sbb[b]], ssem).wait()
                if with_den:
                    pltpu.make_async_copy(pbb[b], den.at[sbb[b]], dsem).wait()
        descs = []
        for b in (0, 1):
            j = blk * 2 + b
            d1 = pltpu.async_copy(sidx_h.at[wid, j], sbb[b], sls[b])
            d2 = pltpu.async_copy(
                src.at[pl.ds(wid * EW + j * CH, CH)], pvb[b], lls[b])
            if with_den:
                d3 = pltpu.async_copy(p2d.at[wid * NCH + j], pbb[b], pls[b])
            else:
                d3 = None
            descs.append((d1, d2, d3))
        for b in (0, 1):
            descs[b][0].wait()
            descs[b][1].wait()
            pltpu.async_copy(pvb[b], acc.at[sbb[b]], ssem, add=True)
            if with_den:
                descs[b][2].wait()
                pltpu.async_copy(pbb[b], den.at[sbb[b]], dsem, add=True)

    def drain_tail(with_den):
        for b in (0, 1):
            pltpu.make_async_copy(pvb[b], acc.at[sbb[b]], ssem).wait()
            if with_den:
                pltpu.make_async_copy(pbb[b], den.at[sbb[b]], dsem).wait()

    run_block(pv_lo, 0, False, True)
    lax.fori_loop(1, NBLK,
                  lambda blk, c: (run_block(pv_lo, blk, True, True), c)[1],
                  0, unroll=False)
    drain_tail(True)
    plsc.subcore_barrier()
    pltpu.sync_copy(acc.at[pl.ds(t * RPT, RPT)],
                    accs.at[cc, 0, pl.ds(t * RPT, RPT)])
    pltpu.sync_copy(den.at[pl.ds(t * RPT, RPT)], dens.at[cc, pl.ds(t * RPT, RPT)])
    pltpu.sync_copy(z2_h, acc.at[pl.ds(t * RPT, RPT)])
    plsc.subcore_barrier()

    run_block(pv_hi, 0, False, False)
    lax.fori_loop(1, NBLK,
                  lambda blk, c: (run_block(pv_hi, blk, True, False), c)[1],
                  0, unroll=False)
    drain_tail(False)
    plsc.subcore_barrier()
    pltpu.sync_copy(acc.at[pl.ds(t * RPT, RPT)],
                    accs.at[cc, 1, pl.ds(t * RPT, RPT)])


def _pad_idx(a, fill):
    ap = jnp.full((NW, EW), fill, jnp.int32)
    ap = ap.at[:, :EW_REAL].set(a.reshape(NW, EW_REAL).astype(jnp.int32))
    return ap.reshape(NW, NCH, CH)


def _direction(q3t, kt, vt, idxq, idxkv, sidx, z2, z1):
    qg, kg, vg = _gather3(q3t.reshape(3 * N, D), kt, vt, idxq, idxkv)
    lo, hi, p2d = _edge_math(qg, kg, vg)
    accs, dens = _scatter(lo, hi, p2d, sidx, z2, z1)
    num = jnp.concatenate([accs[0, 0] + accs[1, 0], accs[0, 1] + accs[1, 1]],
                          axis=1)[:N]
    den = (dens[0] + dens[1])[:N]
    return num / (den[:, None] + 1e-16)


def kernel(params, atoms, edge_index, edge_ids):
    x = params["atom_table"][atoms]
    src, dst = edge_index[0], edge_index[1]
    eid = edge_ids
    inv = 1.0 / math.sqrt(D)

    idxq_r = _pad_idx(eid * N + dst, 0)
    idxq_c = _pad_idx(eid * N + src, 0)
    idxkv_r = _pad_idx(src, 0)
    idxkv_c = _pad_idx(dst, 0)
    sidx_r = _pad_idx(dst, TRASH)
    sidx_c = _pad_idx(src, TRASH)
    z2 = jnp.zeros((RPT, D // 2), jnp.float32)
    z1 = jnp.zeros((RPT,), jnp.float32)

    for l in range(L):
        ws = jnp.stack([
            params["r2c_Wq"][l], params["r2c_Wk"][l], params["r2c_Wv"][l],
            params["c2r_Wq"][l], params["c2r_Wk"][l], params["c2r_Wv"][l],
        ])
        e3s = jnp.stack([
            jnp.tanh(params["edge_table"] @ params["r2c_We"][l]
                     + params["r2c_be"][l]) * inv,
            jnp.tanh(params["edge_table"] @ params["c2r_We"][l]
                     + params["c2r_be"][l]) * inv,
        ])
        q3r, kr, vr, q3c, kc, vc = _prep(x, ws, e3s)
        r2c = _direction(q3r, kr, vr, idxq_r, idxkv_r, sidx_r, z2, z1)
        c2r = _direction(q3c, kc, vc, idxq_c, idxkv_c, sidx_c, z2, z1)
        h = jnp.concatenate([r2c, c2r], axis=-1) @ params["ffn_W"][l]
        y = h + x
        mu = jnp.mean(y, axis=-1, keepdims=True)
        var = jnp.var(y, axis=-1, keepdims=True)
        x = (y - mu) / jnp.sqrt(var + 1e-5) * params["ln_g"][l] + params["ln_b"][l]
    return x
